# baseline (speedup 1.0000x reference)
import functools

import jax
import jax.numpy as jnp
from jax.experimental import pallas as pl
from jax.experimental.pallas import tpu as pltpu

N = 50000
E = 200000
D = 128
ROWS = 2000
CHUNK = 6250


def _mm_kernel(x_ref, so_ref, w_ref, out_ref):
    xs = x_ref[...] * so_ref[0]
    out_ref[0] = jnp.dot(xs, w_ref[0], preferred_element_type=jnp.float32)


def _scaled_matmul(x, so, w):
    return pl.pallas_call(
        _mm_kernel,
        grid=(3, N // ROWS),
        in_specs=[
            pl.BlockSpec((ROWS, D), lambda r, i: (i, 0)),
            pl.BlockSpec((1, ROWS, 1), lambda r, i: (r, i, 0)),
            pl.BlockSpec((1, D, D), lambda r, i: (r, 0, 0)),
        ],
        out_specs=pl.BlockSpec((1, ROWS, D), lambda r, i: (r, i, 0)),
        out_shape=jax.ShapeDtypeStruct((3, N, D), jnp.float32),
    )(x, so, w)


def _scatter_kernel(e_ref, hw_ref, out_ref):
    @pl.when(pl.program_id(0) == 0)
    def _init():
        out_ref[...] = jnp.zeros_like(out_ref)

    def body(i, carry):
        s = e_ref[0, 0, i]
        d = e_ref[0, 1, i]
        out_ref[pl.ds(d, 1), :] += hw_ref[pl.ds(s, 1), :]
        return carry

    jax.lax.fori_loop(0, CHUNK, body, 0)


def _scatter(e, hw):
    e = e.reshape(2, E // CHUNK, CHUNK).swapaxes(0, 1)
    return pl.pallas_call(
        _scatter_kernel,
        grid=(E // CHUNK,),
        in_specs=[
            pl.BlockSpec((1, 2, CHUNK), lambda c: (c, 0, 0), memory_space=pltpu.SMEM),
            pl.BlockSpec((N, D), lambda c: (0, 0)),
        ],
        out_specs=pl.BlockSpec((N, D), lambda c: (0, 0)),
        out_shape=jax.ShapeDtypeStruct((N, D), jnp.float32),
    )(e, hw)


def _combine_kernel(a0_ref, a1_ref, a2_ref, si_ref, b_ref, out_ref, *, relu):
    acc = (a0_ref[...] * si_ref[0] + a1_ref[...] * si_ref[1]
           + a2_ref[...] * si_ref[2] + b_ref[...])
    out_ref[...] = jnp.maximum(acc, 0.0) if relu else acc


def _combine(a0, a1, a2, si, bsum, relu):
    return pl.pallas_call(
        functools.partial(_combine_kernel, relu=relu),
        grid=(N // ROWS,),
        in_specs=[
            pl.BlockSpec((ROWS, D), lambda i: (i, 0)),
            pl.BlockSpec((ROWS, D), lambda i: (i, 0)),
            pl.BlockSpec((ROWS, D), lambda i: (i, 0)),
            pl.BlockSpec((3, ROWS, 1), lambda i: (0, i, 0)),
            pl.BlockSpec((1, D), lambda i: (0, 0)),
        ],
        out_specs=pl.BlockSpec((ROWS, D), lambda i: (i, 0)),
        out_shape=jax.ShapeDtypeStruct((N, D), jnp.float32),
    )(a0, a1, a2, si, bsum)


def _layer(x, edges, so, si, w, bsum, relu):
    hw = _scaled_matmul(x, so, w)
    aggs = [_scatter(edges[r], hw[r]) for r in range(3)]
    return _combine(aggs[0], aggs[1], aggs[2], si, bsum, relu)


def kernel(x, edge_index_r0, edge_index_r1, edge_index_r2,
           W1_r0, b1_r0, W1_r1, b1_r1, W1_r2, b1_r2,
           W2_r0, b2_r0, W2_r1, b2_r1, W2_r2, b2_r2):
    edges = (edge_index_r0, edge_index_r1, edge_index_r2)

    def scale(idx):
        deg = jnp.clip(jnp.bincount(idx, length=N), 1).astype(jnp.float32)
        return deg ** -0.5

    so = jnp.stack([scale(e[0]) for e in edges]).reshape(3, N, 1)
    si = jnp.stack([scale(e[1]) for e in edges]).reshape(3, N, 1)

    w1 = jnp.stack([W1_r0, W1_r1, W1_r2])
    w2 = jnp.stack([W2_r0, W2_r1, W2_r2])
    b1sum = (b1_r0 + b1_r1 + b1_r2).reshape(1, D)
    b2sum = (b2_r0 + b2_r1 + b2_r2).reshape(1, D)

    h = _layer(x, edges, so, si, w1, b1sum, relu=True)
    return _layer(h, edges, so, si, w2, b2sum, relu=False)


# speedup vs baseline: 1.6102x; 1.6102x over previous
import functools

import jax
import jax.numpy as jnp
from jax.experimental import pallas as pl
from jax.experimental.pallas import tpu as pltpu

N = 50000
E = 200000
D = 128
ROWS = 2000
CHUNK = 6250


def _mm_kernel(x_ref, so_ref, w_ref, out_ref):
    xs = x_ref[...] * so_ref[0]
    out_ref[0] = jnp.dot(xs, w_ref[0], preferred_element_type=jnp.float32)


def _scaled_matmul(x, so, w):
    return pl.pallas_call(
        _mm_kernel,
        grid=(3, N // ROWS),
        in_specs=[
            pl.BlockSpec((ROWS, D), lambda r, i: (i, 0)),
            pl.BlockSpec((1, ROWS, 1), lambda r, i: (r, i, 0)),
            pl.BlockSpec((1, D, D), lambda r, i: (r, 0, 0)),
        ],
        out_specs=pl.BlockSpec((1, ROWS, D), lambda r, i: (r, i, 0)),
        out_shape=jax.ShapeDtypeStruct((3, N, D), jnp.float32),
    )(x, so, w)


def _scatter_kernel(e_ref, hw_ref, out_ref):
    @pl.when(pl.program_id(0) == 0)
    def _init():
        out_ref[...] = jnp.zeros_like(out_ref)

    def body(i, carry):
        s = e_ref[0, 0, i]
        d = e_ref[0, 1, i]
        out_ref[pl.ds(d, 1), :] += hw_ref[pl.ds(s, 1), :]
        return carry

    jax.lax.fori_loop(0, CHUNK, body, 0, unroll=8)


def _scatter(e, hw):
    e = e.reshape(2, E // CHUNK, CHUNK).swapaxes(0, 1)
    return pl.pallas_call(
        _scatter_kernel,
        grid=(E // CHUNK,),
        in_specs=[
            pl.BlockSpec((1, 2, CHUNK), lambda c: (c, 0, 0), memory_space=pltpu.SMEM),
            pl.BlockSpec((N, D), lambda c: (0, 0)),
        ],
        out_specs=pl.BlockSpec((N, D), lambda c: (0, 0)),
        out_shape=jax.ShapeDtypeStruct((N, D), jnp.float32),
    )(e, hw)


def _combine_kernel(a0_ref, a1_ref, a2_ref, si_ref, b_ref, out_ref, *, relu):
    acc = (a0_ref[...] * si_ref[0] + a1_ref[...] * si_ref[1]
           + a2_ref[...] * si_ref[2] + b_ref[...])
    out_ref[...] = jnp.maximum(acc, 0.0) if relu else acc


def _combine(a0, a1, a2, si, bsum, relu):
    return pl.pallas_call(
        functools.partial(_combine_kernel, relu=relu),
        grid=(N // ROWS,),
        in_specs=[
            pl.BlockSpec((ROWS, D), lambda i: (i, 0)),
            pl.BlockSpec((ROWS, D), lambda i: (i, 0)),
            pl.BlockSpec((ROWS, D), lambda i: (i, 0)),
            pl.BlockSpec((3, ROWS, 1), lambda i: (0, i, 0)),
            pl.BlockSpec((1, D), lambda i: (0, 0)),
        ],
        out_specs=pl.BlockSpec((ROWS, D), lambda i: (i, 0)),
        out_shape=jax.ShapeDtypeStruct((N, D), jnp.float32),
    )(a0, a1, a2, si, bsum)


def _layer(x, edges, so, si, w, bsum, relu):
    hw = _scaled_matmul(x, so, w)
    aggs = [_scatter(edges[r], hw[r]) for r in range(3)]
    return _combine(aggs[0], aggs[1], aggs[2], si, bsum, relu)


def kernel(x, edge_index_r0, edge_index_r1, edge_index_r2,
           W1_r0, b1_r0, W1_r1, b1_r1, W1_r2, b1_r2,
           W2_r0, b2_r0, W2_r1, b2_r1, W2_r2, b2_r2):
    edges = (edge_index_r0, edge_index_r1, edge_index_r2)

    def scale(idx):
        deg = jnp.clip(jnp.bincount(idx, length=N), 1).astype(jnp.float32)
        return deg ** -0.5

    so = jnp.stack([scale(e[0]) for e in edges]).reshape(3, N, 1)
    si = jnp.stack([scale(e[1]) for e in edges]).reshape(3, N, 1)

    w1 = jnp.stack([W1_r0, W1_r1, W1_r2])
    w2 = jnp.stack([W2_r0, W2_r1, W2_r2])
    b1sum = (b1_r0 + b1_r1 + b1_r2).reshape(1, D)
    b2sum = (b2_r0 + b2_r1 + b2_r2).reshape(1, D)

    h = _layer(x, edges, so, si, w1, b1sum, relu=True)
    return _layer(h, edges, so, si, w2, b2sum, relu=False)
